# Initial kernel scaffold; baseline (speedup 1.0000x reference)
import jax
import jax.numpy as jnp
from jax.experimental import pallas as pl


def kernel(node_features, edge_features, senders, receivers, We1, be1, We2, be2, ge, bbe, Wn1, bn1, Wn2, bn2, gn, bbn):
    raise NotImplementedError("write your pallas kernel here")


# speedup vs baseline: 2.9847x; 2.9847x over previous
import functools

import jax
import jax.numpy as jnp
from jax import lax
from jax.experimental import pallas as pl
from jax.experimental.pallas import tpu as pltpu
from jax.experimental.pallas import tpu_sc as plsc

N = 10000
E = 320000
D = 128

NC = 2
NS = 16
NW = NC * NS
SPAN = E // NW
C = 80
NCH = SPAN // C
N_PAD = 10240
NPS = N_PAD // NS

_EPS = 1e-5


def _ln(y, g, b):
    mu = jnp.mean(y, axis=-1, keepdims=True)
    var = jnp.mean((y - mu) ** 2, axis=-1, keepdims=True)
    return (y - mu) * lax.rsqrt(var + _EPS) * g + b


def _proj_body(x_ref, w_ref, o1_ref, o2_ref, o3_ref):
    x = x_ref[...]
    o1_ref[...] = jnp.dot(x, w_ref[0], preferred_element_type=jnp.float32)
    o2_ref[...] = jnp.dot(x, w_ref[1], preferred_element_type=jnp.float32)
    o3_ref[...] = jnp.dot(x, w_ref[2], preferred_element_type=jnp.float32)


def _proj(x, w3):
    out = jax.ShapeDtypeStruct((N, D), jnp.float32)
    return pl.pallas_call(
        _proj_body,
        out_shape=(out, out, out),
    )(x, w3)


_MESH = plsc.VectorSubcoreMesh(core_axis_name="c", subcore_axis_name="s")


@functools.partial(
    pl.kernel,
    out_type=(jax.ShapeDtypeStruct((E, D), jnp.float32),
              jax.ShapeDtypeStruct((E, D), jnp.float32)),
    mesh=_MESH,
    scratch_types=[
        pltpu.VMEM((C,), jnp.int32),
        pltpu.VMEM((C, D), jnp.float32),
        pltpu.VMEM((C,), jnp.int32),
        pltpu.VMEM((C, D), jnp.float32),
        pltpu.SemaphoreType.DMA,
        pltpu.SemaphoreType.DMA,
    ],
)
def _sc_gather(psrc, pdst, snd, rcv, g1, g2, idx1, buf1, idx2, buf2, s1, s2):
    wid = lax.axis_index("s") * NC + lax.axis_index("c")

    def body(j, carry):
        base = wid * SPAN + j * C
        pltpu.sync_copy(snd.at[pl.ds(base, C)], idx1)
        pltpu.sync_copy(rcv.at[pl.ds(base, C)], idx2)
        cp1 = pltpu.async_copy(psrc.at[idx1], buf1, s1)
        cp2 = pltpu.async_copy(pdst.at[idx2], buf2, s2)
        cp1.wait()
        cp2.wait()
        pltpu.sync_copy(buf1, g1.at[pl.ds(base, C)])
        pltpu.sync_copy(buf2, g2.at[pl.ds(base, C)])
        return carry

    lax.fori_loop(0, NCH, body, 0)


def _edge_body(g1_ref, g2_ref, e_ref, w1_ref, w2_ref, b1_ref, b2_ref,
               ge_ref, bbe_ref, ne_ref, eo_ref):
    eb = e_ref[...]
    x = (jnp.dot(eb, w1_ref[...], preferred_element_type=jnp.float32)
         + g1_ref[...] + g2_ref[...] + b1_ref[...])
    h = jnp.maximum(x, 0.0)
    y = jnp.dot(h, w2_ref[...], preferred_element_type=jnp.float32) + b2_ref[...]
    ne = _ln(y, ge_ref[...], bbe_ref[...])
    ne_ref[...] = ne
    eo_ref[...] = ne + eb


def _edge(g1, g2, e, w1, w2, b1, b2, ge, bbe):
    bm = 2000
    grid = (E // bm,)
    row = pl.BlockSpec((bm, D), lambda i: (i, 0))
    mat = pl.BlockSpec((D, D), lambda i: (0, 0))
    vec = pl.BlockSpec((1, D), lambda i: (0, 0))
    out = jax.ShapeDtypeStruct((E, D), jnp.float32)
    return pl.pallas_call(
        _edge_body,
        grid=grid,
        in_specs=[row, row, row, mat, mat, vec, vec, vec, vec],
        out_specs=(row, row),
        out_shape=(out, out),
    )(g1, g2, e, w1, w2, b1, b2, ge, bbe)


@functools.partial(
    pl.kernel,
    out_type=jax.ShapeDtypeStruct((NC * N_PAD, D), jnp.float32),
    mesh=_MESH,
    scratch_types=[
        pltpu.VMEM((C,), jnp.int32),
        pltpu.VMEM((C, D), jnp.float32),
        pltpu.VMEM_SHARED((N_PAD, D), jnp.float32),
    ],
)
def _sc_scatter(ne, rcv, zeros, part, idx, rows, acc):
    c = lax.axis_index("c")
    s = lax.axis_index("s")
    wid = s * NC + c
    pltpu.sync_copy(zeros.at[pl.ds(s * NPS, NPS)], acc.at[pl.ds(s * NPS, NPS)])
    plsc.subcore_barrier()

    def body(j, carry):
        base = wid * SPAN + j * C
        pltpu.sync_copy(ne.at[pl.ds(base, C)], rows)
        pltpu.sync_copy(rcv.at[pl.ds(base, C)], idx)
        pltpu.sync_copy(rows, acc.at[idx], add=True)
        return carry

    lax.fori_loop(0, NCH, body, 0)
    plsc.subcore_barrier()
    pltpu.sync_copy(acc.at[pl.ds(s * NPS, NPS)],
                    part.at[pl.ds(c * N_PAD + s * NPS, NPS)])


def _node_body(nf_ref, pn_ref, pa_ref, w1_ref, w2_ref, b1_ref, b2_ref,
               g_ref, b_ref, out_ref):
    agg = pa_ref[0] + pa_ref[1]
    x = (pn_ref[...]
         + jnp.dot(agg, w1_ref[...], preferred_element_type=jnp.float32)
         + b1_ref[...])
    h = jnp.maximum(x, 0.0)
    y = jnp.dot(h, w2_ref[...], preferred_element_type=jnp.float32) + b2_ref[...]
    out_ref[...] = _ln(y, g_ref[...], b_ref[...]) + nf_ref[...]


def _node(nf, pn, part, w1a, w2, b1, b2, g, b):
    bm = 2000
    grid = (N // bm,)
    row = pl.BlockSpec((bm, D), lambda i: (i, 0))
    pa = pl.BlockSpec((2, bm, D), lambda i: (0, i, 0))
    mat = pl.BlockSpec((D, D), lambda i: (0, 0))
    vec = pl.BlockSpec((1, D), lambda i: (0, 0))
    return pl.pallas_call(
        _node_body,
        grid=grid,
        in_specs=[row, row, pa, mat, mat, vec, vec, vec, vec],
        out_specs=row,
        out_shape=jax.ShapeDtypeStruct((N, D), jnp.float32),
    )(nf, pn, part, w1a, w2, b1, b2, g, b)


def kernel(node_features, edge_features, senders, receivers,
           We1, be1, We2, be2, ge, bbe,
           Wn1, bn1, Wn2, bn2, gn, bbn):
    w3 = jnp.stack([We1[:D], We1[D:2 * D], Wn1[:D]])
    psrc, pdst, pn = _proj(node_features, w3)

    g1, g2 = _sc_gather(psrc, pdst, senders, receivers)

    ne, eo = _edge(g1, g2, edge_features, We1[2 * D:], We2,
                   be1[None], be2[None], ge[None], bbe[None])

    zeros = jnp.zeros((N_PAD, D), jnp.float32)
    part = _sc_scatter(ne, receivers, zeros).reshape(NC, N_PAD, D)

    node_out = _node(node_features, pn, part, Wn1[D:], Wn2,
                     bn1[None], bn2[None], gn[None], bbn[None])
    return node_out, eo
